# baseline (speedup 1.0000x reference)
import functools

import jax
import jax.numpy as jnp
import numpy as np
from jax import lax
from jax.experimental import pallas as pl
from jax.experimental.pallas import tpu as pltpu
from jax.experimental.pallas import tpu_sc as plsc

_B, _T, _D = 4, 4096, 1024
_V = 1000000
_NGRAM_HEADS = 4
_MAX_NGRAM = 3
_E = 16
_NUM_TABLES = (_MAX_NGRAM - 1) * _NGRAM_HEADS
_N = _B * _T
_TB = 512
_NW = 32
_RW = _N // _NW
_CH = 128
_NCHUNK = _NUM_TABLES * (_RW // _CH)


def _hash_indices(input_ids, hash_mult):
    ids = input_ids.astype(jnp.int64)
    idx_list = []
    for n in range(2, _MAX_NGRAM + 1):
        tokens = [ids]
        for offset in range(1, n):
            pad = jnp.zeros((ids.shape[0], offset), dtype=ids.dtype)
            tokens.append(jnp.concatenate([pad, ids[:, :-offset]], axis=1))
        h = tokens[0] * hash_mult[0]
        for i in range(1, n):
            h = h ^ (tokens[i] * hash_mult[i])
        for head in range(_NGRAM_HEADS):
            idx_list.append((h + head * 7919) % _V)
    return jnp.stack(idx_list).reshape(_NUM_TABLES, _N).astype(jnp.int32)


_VB = 16384


_HT = _NUM_TABLES // 2
_HC = _HT * _E


def _tr_body(in_ref, out_ref):
    x = in_ref[...]
    y = x.reshape(_HC, _VB)
    z = jnp.swapaxes(y, 0, 1)
    z3 = z.reshape(_VB // 2, 2, _HC)
    out_ref[...] = jnp.concatenate([z3[:, 0], z3[:, 1]], axis=1)


def _tc_transpose(tt_h):
    return pl.pallas_call(
        _tr_body,
        grid=((_V + _VB - 1) // _VB,),
        in_specs=[pl.BlockSpec((_HT, _E, _VB), lambda i: (_Z, _Z, i))],
        out_specs=pl.BlockSpec((_VB // 2, 2 * _HC), lambda i: (i, _Z)),
        out_shape=jax.ShapeDtypeStruct((_V // 2, 2 * _HC), jnp.float32),
    )(tt_h)


_NCH_H = _HT * (_RW // _CH)


def _sc_gather(mt_h, idx_hi, idx_lo):
    mesh = plsc.VectorSubcoreMesh(core_axis_name="c", subcore_axis_name="s")
    nq = _RW // _CH

    @functools.partial(
        pl.kernel,
        mesh=mesh,
        out_type=jax.ShapeDtypeStruct((_N, _HC), jnp.float32),
        compiler_params=pltpu.CompilerParams(needs_layout_passes=False),
        scratch_types=[
            pltpu.VMEM((_NCH_H, _CH), jnp.int32),
            pltpu.VMEM((_NCH_H, 8, 16), jnp.int32),
            pltpu.VMEM((_CH, 2 * _HC), jnp.float32),
            pltpu.VMEM((_CH, _HC), jnp.float32),
            pltpu.SemaphoreType.DMA,
        ],
    )
    def gather_k(mt_hbm, idxhi_hbm, idxlo_hbm, out_hbm, idx_v, idxlo_v,
                 buf_v, asm_v, sem):
        wid = lax.axis_index("s") * 2 + lax.axis_index("c")
        pltpu.sync_copy(idxhi_hbm.at[wid], idx_v)
        pltpu.sync_copy(idxlo_hbm.at[wid], idxlo_v)
        row_base = wid * _RW
        zeros16 = jnp.zeros((16,), jnp.int32)

        def table_body(t, q):
            c = q * jnp.int32(_HT) + t
            pltpu.async_copy(mt_hbm.at[idx_v.at[c]], buf_v, sem).wait()
            base = t * jnp.int32(16)

            def group_body(g, _):
                rows = g * jnp.int32(16) + lax.iota(jnp.int32, 16)
                off = idxlo_v[c, g]
                for l in range(16):
                    ld_col = off + base + jnp.int32(l)
                    vals = plsc.load_gather(buf_v, [rows, ld_col])
                    st_col = base + jnp.int32(l) + zeros16
                    plsc.store_scatter(asm_v, [rows, st_col], vals)
                return _

            lax.fori_loop(jnp.int32(0), jnp.int32(_CH // 16), group_body,
                          jnp.int32(0))
            return q

        def q_body(q, _):
            lax.fori_loop(jnp.int32(0), jnp.int32(_HT), table_body, q)
            pltpu.sync_copy(asm_v, out_hbm.at[pl.ds(row_base + q * _CH, _CH)])
            return _

        lax.fori_loop(jnp.int32(0), jnp.int32(nq), q_body, jnp.int32(0))

    return gather_k(mt_h, idx_hi, idx_lo)


def _tc_body(elo_ref, ehi_ref, hid_ref, wk_ref, wv_ref, bk_ref, bv_ref,
             nk_ref, nq_ref, cw_ref, out_ref, carry):
    tb = pl.program_id(1)
    elo = elo_ref[0]
    ehi = ehi_ref[0]
    h = hid_ref[0]
    dn = (((1,), (0,)), ((), ()))
    key = (lax.dot_general(elo, wk_ref[0:_HC], dn,
                           precision=lax.Precision.HIGHEST,
                           preferred_element_type=jnp.float32)
           + lax.dot_general(ehi, wk_ref[_HC:2 * _HC], dn,
                             precision=lax.Precision.HIGHEST,
                             preferred_element_type=jnp.float32)
           + bk_ref[0])
    norm_k = lax.rsqrt(jnp.mean(key * key, axis=1, keepdims=True) + 1e-6)
    norm_h = lax.rsqrt(jnp.mean(h * h, axis=1, keepdims=True) + 1e-6)
    a = key * norm_k * nk_ref[0]
    q = h * norm_h * nq_ref[0]
    g = jnp.sum(a * q, axis=1, keepdims=True) * np.float32(1.0 / np.sqrt(_D))
    s = jnp.sqrt(jnp.clip(jnp.abs(g), np.float32(1e-6), None))
    g = jnp.where(g < 0, -s, jnp.where(g > 0, s, jnp.float32(0.0)))
    g = jax.nn.sigmoid(g)
    value = g * (lax.dot_general(elo, wv_ref[0:_HC], dn,
                                 precision=lax.Precision.HIGHEST,
                                 preferred_element_type=jnp.float32)
                 + lax.dot_general(ehi, wv_ref[_HC:2 * _HC], dn,
                                   precision=lax.Precision.HIGHEST,
                                   preferred_element_type=jnp.float32)
                 + bv_ref[0])

    @pl.when(tb == 0)
    def _():
        carry[...] = jnp.zeros_like(carry)

    ext = jnp.concatenate([carry[5:8], value], axis=0)
    cw = cw_ref[...]
    vc = (ext[0:_TB] * cw[0] + ext[1:_TB + 1] * cw[1]
          + ext[2:_TB + 2] * cw[2] + ext[3:_TB + 3] * cw[3])
    out_ref[0] = value + vc * jax.nn.sigmoid(vc)
    carry[...] = value[_TB - 8:_TB]


_Z = np.int32(0)


def _tc_call(elo3, ehi3, hidden, Wk, Wv, bk2, bv2, nk2, nq2, cwT):
    nt = _T // _TB
    return pl.pallas_call(
        _tc_body,
        grid=(_B, nt),
        in_specs=[
            pl.BlockSpec((1, _TB, _HC), lambda b, t: (b, t, _Z)),
            pl.BlockSpec((1, _TB, _HC), lambda b, t: (b, t, _Z)),
            pl.BlockSpec((1, _TB, _D), lambda b, t: (b, t, _Z)),
            pl.BlockSpec((_NUM_TABLES * _E, _D), lambda b, t: (_Z, _Z)),
            pl.BlockSpec((_NUM_TABLES * _E, _D), lambda b, t: (_Z, _Z)),
            pl.BlockSpec((1, _D), lambda b, t: (_Z, _Z)),
            pl.BlockSpec((1, _D), lambda b, t: (_Z, _Z)),
            pl.BlockSpec((1, _D), lambda b, t: (_Z, _Z)),
            pl.BlockSpec((1, _D), lambda b, t: (_Z, _Z)),
            pl.BlockSpec((4, _D), lambda b, t: (_Z, _Z)),
        ],
        out_specs=pl.BlockSpec((1, _TB, _D), lambda b, t: (b, t, _Z)),
        out_shape=jax.ShapeDtypeStruct((_B, _T, _D), jnp.float32),
        scratch_shapes=[pltpu.VMEM((8, _D), jnp.float32)],
    )(elo3, ehi3, hidden, Wk, Wv, bk2, bv2, nk2, nq2, cwT)


def kernel(hidden_states, input_ids, emb_tables, Wk, bk, Wv, bv, nk_w, nq_w,
           conv_w, hash_mult):
    idx = _hash_indices(input_ids, hash_mult)
    tt = jnp.transpose(emb_tables, (0, 2, 1))
    embs = []
    for half in range(2):
        idx_h = (idx[half * _HT:(half + 1) * _HT]
                 .reshape(_HT, _NW, _RW // _CH, _CH)
                 .transpose(1, 2, 0, 3)
                 .reshape(_NW, _NCH_H, _CH))
        idx_hi = idx_h // 2
        idx_lo = ((idx_h % 2) * _HC).reshape(_NW, _NCH_H, 8, 16)
        mt_h = _tc_transpose(tt[half * _HT:(half + 1) * _HT])
        embs.append(_sc_gather(mt_h, idx_hi, idx_lo).reshape(_B, _T, _HC))
    out = _tc_call(
        embs[0],
        embs[1],
        hidden_states,
        Wk,
        Wv,
        bk.reshape(1, _D),
        bv.reshape(1, _D),
        nk_w.reshape(1, _D),
        nq_w.reshape(1, _D),
        conv_w.T,
    )
    return out


# speedup vs baseline: 2.5991x; 2.5991x over previous
import functools

import jax
import jax.numpy as jnp
import numpy as np
from jax import lax
from jax.experimental import pallas as pl
from jax.experimental.pallas import tpu as pltpu
from jax.experimental.pallas import tpu_sc as plsc

_B, _T, _D = 4, 4096, 1024
_V = 1000000
_NGRAM_HEADS = 4
_MAX_NGRAM = 3
_E = 16
_NUM_TABLES = (_MAX_NGRAM - 1) * _NGRAM_HEADS
_C = _NUM_TABLES * _E
_N = _B * _T
_TB = 512
_NW = 32
_RW = _N // _NW
_CH = 128
_NCHUNK = _NUM_TABLES * (_RW // _CH)
_Z = np.int32(0)


def _hash_indices(input_ids, hash_mult):
    ids = input_ids.astype(jnp.int64)
    idx_list = []
    for n in range(2, _MAX_NGRAM + 1):
        tokens = [ids]
        for offset in range(1, n):
            pad = jnp.zeros((ids.shape[0], offset), dtype=ids.dtype)
            tokens.append(jnp.concatenate([pad, ids[:, :-offset]], axis=1))
        h = tokens[0] * hash_mult[0]
        for i in range(1, n):
            h = h ^ (tokens[i] * hash_mult[i])
        for head in range(_NGRAM_HEADS):
            idx_list.append((h + head * 7919) % _V)
    return jnp.stack(idx_list).reshape(_NUM_TABLES, _N).astype(jnp.int32)


_VB = 16384


def _tr_body(in_ref, out_ref):
    x = in_ref[...]
    y = x.reshape(_C, _VB)
    out_ref[...] = jnp.swapaxes(y, 0, 1)


def _tc_transpose(tt):
    return pl.pallas_call(
        _tr_body,
        grid=((_V + _VB - 1) // _VB,),
        in_specs=[pl.BlockSpec((_NUM_TABLES, _E, _VB), lambda i: (_Z, _Z, i))],
        out_specs=pl.BlockSpec((_VB, _C), lambda i: (i, _Z)),
        out_shape=jax.ShapeDtypeStruct((_V, _C), jnp.float32),
    )(tt)


def _sc_gather(mt, idx_arr):
    mesh = plsc.VectorSubcoreMesh(core_axis_name="c", subcore_axis_name="s")

    @functools.partial(
        pl.kernel,
        mesh=mesh,
        out_type=jax.ShapeDtypeStruct((_N, _C), jnp.float32),
        compiler_params=pltpu.CompilerParams(needs_layout_passes=False),
        scratch_types=[
            pltpu.VMEM((_NCHUNK, _CH), jnp.int32),
            pltpu.VMEM((_CH, _C), jnp.float32),
            pltpu.VMEM((_CH, _C), jnp.float32),
            pltpu.VMEM((_CH, _C), jnp.float32),
            pltpu.SemaphoreType.DMA,
            pltpu.SemaphoreType.DMA,
        ],
    )
    def gather_k(mt_hbm, idx_hbm, out_hbm, idx_v, buf0, buf1, asm_v,
                 sem0, sem1):
        wid = lax.axis_index("s") * 2 + lax.axis_index("c")
        pltpu.sync_copy(idx_hbm.at[wid], idx_v)
        row_base = wid * _RW
        zeros16 = jnp.zeros((16,), jnp.int32)

        def start(c, buf, sem):
            return pltpu.async_copy(mt_hbm.at[idx_v.at[c]], buf, sem)

        def extract(c, buf):
            q = lax.div(c, jnp.int32(_NUM_TABLES))
            base = (c - q * jnp.int32(_NUM_TABLES)) * jnp.int32(16)

            def group_body(g, _):
                rows = g * jnp.int32(16) + lax.iota(jnp.int32, 16)
                for l in range(16):
                    colv = base + jnp.int32(l) + zeros16
                    vals = plsc.load_gather(buf, [rows, colv])
                    plsc.store_scatter(asm_v, [rows, colv], vals)
                return _

            lax.fori_loop(jnp.int32(0), jnp.int32(_CH // 16), group_body,
                          jnp.int32(0))

        def flush(q):
            pltpu.sync_copy(asm_v, out_hbm.at[pl.ds(row_base + q * _CH, _CH)])

        start(jnp.int32(0), buf0, sem0)

        def pair_body(k, carry):
            a = k * jnp.int32(2)
            b = a + jnp.int32(1)
            pltpu.make_async_copy(mt_hbm.at[idx_v.at[a]], buf0, sem0).wait()
            start(b, buf1, sem1)
            extract(a, buf0)
            pltpu.make_async_copy(mt_hbm.at[idx_v.at[b]], buf1, sem1).wait()

            @pl.when(b < jnp.int32(_NCHUNK - 1))
            def _():
                start(b + jnp.int32(1), buf0, sem0)

            extract(b, buf1)

            @pl.when(lax.rem(b, jnp.int32(8)) == jnp.int32(7))
            def _():
                flush(lax.div(b, jnp.int32(_NUM_TABLES)))

            return carry

        lax.fori_loop(jnp.int32(0), jnp.int32(_NCHUNK // 2), pair_body,
                      jnp.int32(0))

    return gather_k(mt, idx_arr)


def _tc_body(emb_ref, hid_ref, wk_ref, wv_ref, bk_ref, bv_ref, nk_ref,
             nq_ref, cw_ref, out_ref, carry):
    tb = pl.program_id(1)
    e = emb_ref[0]
    h = hid_ref[0]
    dn = (((1,), (0,)), ((), ()))
    key = lax.dot_general(e, wk_ref[...], dn,
                          preferred_element_type=jnp.float32) + bk_ref[0]
    norm_k = lax.rsqrt(jnp.mean(key * key, axis=1, keepdims=True) + 1e-6)
    norm_h = lax.rsqrt(jnp.mean(h * h, axis=1, keepdims=True) + 1e-6)
    a = key * norm_k * nk_ref[0]
    q = h * norm_h * nq_ref[0]
    g = jnp.sum(a * q, axis=1, keepdims=True) * np.float32(1.0 / np.sqrt(_D))
    s = jnp.sqrt(jnp.clip(jnp.abs(g), np.float32(1e-6), None))
    g = jnp.where(g < 0, -s, jnp.where(g > 0, s, jnp.float32(0.0)))
    g = jax.nn.sigmoid(g)
    value = g * (lax.dot_general(e, wv_ref[...], dn,
                                 preferred_element_type=jnp.float32)
                 + bv_ref[0])

    @pl.when(tb == 0)
    def _():
        carry[...] = jnp.zeros_like(carry)

    ext = jnp.concatenate([carry[5:8], value], axis=0)
    cw = cw_ref[...]
    vc = (ext[0:_TB] * cw[0] + ext[1:_TB + 1] * cw[1]
          + ext[2:_TB + 2] * cw[2] + ext[3:_TB + 3] * cw[3])
    out_ref[0] = value + vc * jax.nn.sigmoid(vc)
    carry[...] = value[_TB - 8:_TB]


def _tc_call(emb3, hidden, Wk, Wv, bk2, bv2, nk2, nq2, cwT):
    nt = _T // _TB
    return pl.pallas_call(
        _tc_body,
        grid=(_B, nt),
        in_specs=[
            pl.BlockSpec((1, _TB, _C), lambda b, t: (b, t, _Z)),
            pl.BlockSpec((1, _TB, _D), lambda b, t: (b, t, _Z)),
            pl.BlockSpec((_C, _D), lambda b, t: (_Z, _Z)),
            pl.BlockSpec((_C, _D), lambda b, t: (_Z, _Z)),
            pl.BlockSpec((1, _D), lambda b, t: (_Z, _Z)),
            pl.BlockSpec((1, _D), lambda b, t: (_Z, _Z)),
            pl.BlockSpec((1, _D), lambda b, t: (_Z, _Z)),
            pl.BlockSpec((1, _D), lambda b, t: (_Z, _Z)),
            pl.BlockSpec((4, _D), lambda b, t: (_Z, _Z)),
        ],
        out_specs=pl.BlockSpec((1, _TB, _D), lambda b, t: (b, t, _Z)),
        out_shape=jax.ShapeDtypeStruct((_B, _T, _D), jnp.float32),
        scratch_shapes=[pltpu.VMEM((8, _D), jnp.float32)],
    )(emb3, hidden, Wk, Wv, bk2, bv2, nk2, nq2, cwT)


def kernel(hidden_states, input_ids, emb_tables, Wk, bk, Wv, bv, nk_w, nq_w,
           conv_w, hash_mult):
    idx = _hash_indices(input_ids, hash_mult)
    idx_arr = (idx.reshape(_NUM_TABLES, _NW, _RW // _CH, _CH)
               .transpose(1, 2, 0, 3)
               .reshape(_NW, _NCHUNK, _CH))
    mt = _tc_transpose(jnp.transpose(emb_tables, (0, 2, 1)))
    emb_flat = _sc_gather(mt, idx_arr)
    emb3 = emb_flat.reshape(_B, _T, _C)
    out = _tc_call(
        emb3,
        hidden_states,
        Wk,
        Wv,
        bk.reshape(1, _D),
        bv.reshape(1, _D),
        nk_w.reshape(1, _D),
        nq_w.reshape(1, _D),
        conv_w.T,
    )
    return out


# speedup vs baseline: 2.6174x; 1.0071x over previous
import functools

import jax
import jax.numpy as jnp
import numpy as np
from jax import lax
from jax.experimental import pallas as pl
from jax.experimental.pallas import tpu as pltpu
from jax.experimental.pallas import tpu_sc as plsc

_B, _T, _D = 4, 4096, 1024
_V = 1000000
_NGRAM_HEADS = 4
_MAX_NGRAM = 3
_E = 16
_NUM_TABLES = (_MAX_NGRAM - 1) * _NGRAM_HEADS
_C = _NUM_TABLES * _E
_N = _B * _T
_TB = 1024
_NW = 32
_RW = _N // _NW
_CH = 128
_NCHUNK = _NUM_TABLES * (_RW // _CH)
_Z = np.int32(0)


def _hash_indices(input_ids, hash_mult):
    ids = input_ids.astype(jnp.int64)
    idx_list = []
    for n in range(2, _MAX_NGRAM + 1):
        tokens = [ids]
        for offset in range(1, n):
            pad = jnp.zeros((ids.shape[0], offset), dtype=ids.dtype)
            tokens.append(jnp.concatenate([pad, ids[:, :-offset]], axis=1))
        h = tokens[0] * hash_mult[0]
        for i in range(1, n):
            h = h ^ (tokens[i] * hash_mult[i])
        for head in range(_NGRAM_HEADS):
            idx_list.append((h + head * 7919) % _V)
    return jnp.stack(idx_list).reshape(_NUM_TABLES, _N).astype(jnp.int32)


_VB = 16384


def _tr_body(in_ref, out_ref):
    x = in_ref[...]
    y = x.reshape(_C, _VB)
    out_ref[...] = jnp.swapaxes(y, 0, 1)


def _tc_transpose(tt):
    return pl.pallas_call(
        _tr_body,
        grid=((_V + _VB - 1) // _VB,),
        in_specs=[pl.BlockSpec((_NUM_TABLES, _E, _VB), lambda i: (_Z, _Z, i))],
        out_specs=pl.BlockSpec((_VB, _C), lambda i: (i, _Z)),
        out_shape=jax.ShapeDtypeStruct((_V, _C), jnp.float32),
    )(tt)


def _sc_gather(mt, idx_arr):
    mesh = plsc.VectorSubcoreMesh(core_axis_name="c", subcore_axis_name="s")

    @functools.partial(
        pl.kernel,
        mesh=mesh,
        out_type=jax.ShapeDtypeStruct((_N, _C), jnp.float32),
        compiler_params=pltpu.CompilerParams(needs_layout_passes=False),
        scratch_types=[
            pltpu.VMEM((_NCHUNK, _CH), jnp.int32),
            pltpu.VMEM((_CH, _C), jnp.float32),
            pltpu.VMEM((_CH, _C), jnp.float32),
            pltpu.VMEM((_CH, _C), jnp.float32),
            pltpu.SemaphoreType.DMA,
            pltpu.SemaphoreType.DMA,
        ],
    )
    def gather_k(mt_hbm, idx_hbm, out_hbm, idx_v, buf0, buf1, asm_v,
                 sem0, sem1):
        wid = lax.axis_index("s") * 2 + lax.axis_index("c")
        pltpu.sync_copy(idx_hbm.at[wid], idx_v)
        row_base = wid * _RW
        zeros16 = jnp.zeros((16,), jnp.int32)

        def start(c, buf, sem):
            return pltpu.async_copy(mt_hbm.at[idx_v.at[c]], buf, sem)

        def extract(c, buf):
            q = lax.div(c, jnp.int32(_NUM_TABLES))
            base = (c - q * jnp.int32(_NUM_TABLES)) * jnp.int32(16)

            def group_body(g, _):
                rows = g * jnp.int32(16) + lax.iota(jnp.int32, 16)
                for l in range(16):
                    colv = base + jnp.int32(l) + zeros16
                    vals = plsc.load_gather(buf, [rows, colv])
                    plsc.store_scatter(asm_v, [rows, colv], vals)
                return _

            lax.fori_loop(jnp.int32(0), jnp.int32(_CH // 16), group_body,
                          jnp.int32(0))

        def flush(q):
            pltpu.sync_copy(asm_v, out_hbm.at[pl.ds(row_base + q * _CH, _CH)])

        start(jnp.int32(0), buf0, sem0)

        def pair_body(k, carry):
            a = k * jnp.int32(2)
            b = a + jnp.int32(1)
            pltpu.make_async_copy(mt_hbm.at[idx_v.at[a]], buf0, sem0).wait()
            start(b, buf1, sem1)
            extract(a, buf0)
            pltpu.make_async_copy(mt_hbm.at[idx_v.at[b]], buf1, sem1).wait()

            @pl.when(b < jnp.int32(_NCHUNK - 1))
            def _():
                start(b + jnp.int32(1), buf0, sem0)

            extract(b, buf1)

            @pl.when(lax.rem(b, jnp.int32(8)) == jnp.int32(7))
            def _():
                flush(lax.div(b, jnp.int32(_NUM_TABLES)))

            return carry

        lax.fori_loop(jnp.int32(0), jnp.int32(_NCHUNK // 2), pair_body,
                      jnp.int32(0))

    return gather_k(mt, idx_arr)


def _tc_body(emb_ref, hid_ref, wk_ref, wv_ref, bk_ref, bv_ref, nk_ref,
             nq_ref, cw_ref, out_ref, carry):
    tb = pl.program_id(1)
    e = emb_ref[0]
    h = hid_ref[0]
    dn = (((1,), (0,)), ((), ()))
    key = lax.dot_general(e, wk_ref[...], dn,
                          preferred_element_type=jnp.float32) + bk_ref[0]
    norm_k = lax.rsqrt(jnp.mean(key * key, axis=1, keepdims=True) + 1e-6)
    norm_h = lax.rsqrt(jnp.mean(h * h, axis=1, keepdims=True) + 1e-6)
    a = key * norm_k * nk_ref[0]
    q = h * norm_h * nq_ref[0]
    g = jnp.sum(a * q, axis=1, keepdims=True) * np.float32(1.0 / np.sqrt(_D))
    s = jnp.sqrt(jnp.clip(jnp.abs(g), np.float32(1e-6), None))
    g = jnp.where(g < 0, -s, jnp.where(g > 0, s, jnp.float32(0.0)))
    g = jax.nn.sigmoid(g)
    value = g * (lax.dot_general(e, wv_ref[...], dn,
                                 preferred_element_type=jnp.float32)
                 + bv_ref[0])

    @pl.when(tb == 0)
    def _():
        carry[...] = jnp.zeros_like(carry)

    ext = jnp.concatenate([carry[5:8], value], axis=0)
    cw = cw_ref[...]
    vc = (ext[0:_TB] * cw[0] + ext[1:_TB + 1] * cw[1]
          + ext[2:_TB + 2] * cw[2] + ext[3:_TB + 3] * cw[3])
    out_ref[0] = value + vc * jax.nn.sigmoid(vc)
    carry[...] = value[_TB - 8:_TB]


def _tc_call(emb3, hidden, Wk, Wv, bk2, bv2, nk2, nq2, cwT):
    nt = _T // _TB
    return pl.pallas_call(
        _tc_body,
        grid=(_B, nt),
        in_specs=[
            pl.BlockSpec((1, _TB, _C), lambda b, t: (b, t, _Z)),
            pl.BlockSpec((1, _TB, _D), lambda b, t: (b, t, _Z)),
            pl.BlockSpec((_C, _D), lambda b, t: (_Z, _Z)),
            pl.BlockSpec((_C, _D), lambda b, t: (_Z, _Z)),
            pl.BlockSpec((1, _D), lambda b, t: (_Z, _Z)),
            pl.BlockSpec((1, _D), lambda b, t: (_Z, _Z)),
            pl.BlockSpec((1, _D), lambda b, t: (_Z, _Z)),
            pl.BlockSpec((1, _D), lambda b, t: (_Z, _Z)),
            pl.BlockSpec((4, _D), lambda b, t: (_Z, _Z)),
        ],
        out_specs=pl.BlockSpec((1, _TB, _D), lambda b, t: (b, t, _Z)),
        out_shape=jax.ShapeDtypeStruct((_B, _T, _D), jnp.float32),
        scratch_shapes=[pltpu.VMEM((8, _D), jnp.float32)],
    )(emb3, hidden, Wk, Wv, bk2, bv2, nk2, nq2, cwT)


def kernel(hidden_states, input_ids, emb_tables, Wk, bk, Wv, bv, nk_w, nq_w,
           conv_w, hash_mult):
    idx = _hash_indices(input_ids, hash_mult)
    idx_arr = (idx.reshape(_NUM_TABLES, _NW, _RW // _CH, _CH)
               .transpose(1, 2, 0, 3)
               .reshape(_NW, _NCHUNK, _CH))
    mt = _tc_transpose(jnp.transpose(emb_tables, (0, 2, 1)))
    emb_flat = _sc_gather(mt, idx_arr)
    emb3 = emb_flat.reshape(_B, _T, _C)
    out = _tc_call(
        emb3,
        hidden_states,
        Wk,
        Wv,
        bk.reshape(1, _D),
        bv.reshape(1, _D),
        nk_w.reshape(1, _D),
        nq_w.reshape(1, _D),
        conv_w.T,
    )
    return out
